# Initial kernel scaffold; baseline (speedup 1.0000x reference)
import jax
import jax.numpy as jnp
from jax.experimental import pallas as pl


def kernel(x, pos_table):
    raise NotImplementedError("write your pallas kernel here")


# speedup vs baseline: 1.5481x; 1.5481x over previous
import jax
import jax.numpy as jnp
from jax.experimental import pallas as pl

BATCH = 4
MAXLEN = 8192
EMBED_DIM = 2048

SEQ_BLK = 512


def _add_kernel(x_ref, pos_ref, o_ref):
    o_ref[...] = x_ref[...] + pos_ref[...]


def kernel(x, pos_table):
    grid = (MAXLEN // SEQ_BLK, BATCH)
    return pl.pallas_call(
        _add_kernel,
        grid=grid,
        in_specs=[
            pl.BlockSpec((1, SEQ_BLK, EMBED_DIM), lambda s, b: (b, s, 0)),
            pl.BlockSpec((SEQ_BLK, EMBED_DIM), lambda s, b: (s, 0)),
        ],
        out_specs=pl.BlockSpec((1, SEQ_BLK, EMBED_DIM), lambda s, b: (b, s, 0)),
        out_shape=jax.ShapeDtypeStruct(x.shape, x.dtype),
    )(x, pos_table)


# speedup vs baseline: 1.5946x; 1.0301x over previous
import jax
import jax.numpy as jnp
from jax.experimental import pallas as pl

BATCH = 4
MAXLEN = 8192
EMBED_DIM = 2048

SEQ_BLK = 256


def _add_kernel(x_ref, pos_ref, o_ref):
    o_ref[...] = x_ref[...] + pos_ref[...]


def kernel(x, pos_table):
    grid = (MAXLEN // SEQ_BLK,)
    return pl.pallas_call(
        _add_kernel,
        grid=grid,
        in_specs=[
            pl.BlockSpec((BATCH, SEQ_BLK, EMBED_DIM), lambda s: (0, s, 0)),
            pl.BlockSpec((SEQ_BLK, EMBED_DIM), lambda s: (s, 0)),
        ],
        out_specs=pl.BlockSpec((BATCH, SEQ_BLK, EMBED_DIM), lambda s: (0, s, 0)),
        out_shape=jax.ShapeDtypeStruct(x.shape, x.dtype),
    )(x, pos_table)
